# baseline (speedup 1.0000x reference)
import functools

import numpy as np

import jax
import jax.numpy as jnp
from jax import lax
from jax.experimental import pallas as pl
from jax.experimental.pallas import tpu as pltpu
from jax.experimental.pallas import tpu_sc as plsc

F = 128
C = 144
TW = 160
NC = 2
NS = 16
LANES = 16
BE = 40
G = 50
EPS = 1e-9
DEG_EPS = 1e-6
_IL = plsc.PackFormat.INTERLEAVED


def _edge_pass(table, src2, dst2, zeros):
    n = table.shape[0]
    nrows = dst2.shape[0]
    n_tiles = NC * NS
    rows_pt = nrows // n_tiles
    ngrp = rows_pt // G
    rpt = n // NS

    mesh = plsc.VectorSubcoreMesh(
        core_axis_name="c", subcore_axis_name="s",
        num_cores=NC, num_subcores=NS)

    @functools.partial(
        pl.kernel,
        out_type=jax.ShapeDtypeStruct((NC, n, C), jnp.float32),
        mesh=mesh,
        scratch_types=[
            pltpu.VMEM((G, 2 * BE), jnp.int32),
            pltpu.VMEM((G, BE), jnp.int32),
            pltpu.VMEM((2 * BE, TW), jnp.bfloat16),
            pltpu.VMEM((2 * BE, TW), jnp.bfloat16),
            pltpu.VMEM((BE, C), jnp.float32),
            pltpu.VMEM((BE, C), jnp.float32),
            pltpu.VMEM_SHARED((n, C), jnp.float32),
            pltpu.SemaphoreType.DMA,
            pltpu.SemaphoreType.DMA,
            pltpu.SemaphoreType.DMA,
            pltpu.SemaphoreType.DMA,
        ],
        compiler_params=pltpu.CompilerParams(
            use_tc_tiling_on_sc=False, needs_layout_passes=False),
    )
    def edge_kernel(table_h, src_h, dst_h, zeros_h, out_h,
                    midx, didx, xxa, xxb, ora, orb, acc,
                    gsa, gsb, ssa, ssb):
        cid = lax.axis_index("c")
        sid = lax.axis_index("s")
        wid = sid * NC + cid

        lane = lax.iota(jnp.int32, LANES)
        unit = jnp.where(lane == 0, 1.0, 0.0).astype(jnp.float32)

        pltpu.sync_copy(zeros_h, acc.at[pl.ds(sid * rpt, rpt)])
        plsc.subcore_barrier()

        @pl.loop(0, BE)
        def _(ed):
            ora[ed, pl.ds(F, LANES)] = unit
            orb[ed, pl.ds(F, LANES)] = unit

        def start_gathers(b, xx, sem):
            pltpu.async_copy(table_h.at[midx.at[b]], xx, sem)

        def wait_gathers(b, xx, sem):
            pltpu.make_async_copy(table_h.at[midx.at[b]], xx, sem).wait()

        def wait_scatter(orow, sem):
            pltpu.make_async_copy(orow, acc.at[didx.at[0]], sem).wait()

        def compute(xx, orow):
            @plsc.parallel_loop(0, BE, unroll=4)
            def _(ed):
                xjc = [xx[ed, pl.ds(c * 32, 32)] for c in range(4)]
                xic = [xx[ed + BE, pl.ds(c * 32, 32)] for c in range(4)]
                p = [xic[c] * xjc[c] for c in range(4)]
                s = (p[0] + p[1]) + (p[2] + p[3])
                lo, hi = plsc.unpack(s, format=_IL)
                dot = jnp.sum(lo + hi)
                dv = jnp.full((LANES,), dot, jnp.float32)
                nprod = xx[ed + BE, pl.ds(F, 32)] * xx[ed, pl.ds(F, 32)]
                den, _ = plsc.unpack(nprod, format=_IL)
                q = dv * dv / jnp.maximum(den, EPS)
                wv = jnp.exp(q - 1.0)
                for c in range(4):
                    a, b = plsc.unpack(xjc[c], format=_IL)
                    orow[ed, pl.ds(c * 32, LANES)] = wv * a
                    orow[ed, pl.ds(c * 32 + LANES, LANES)] = wv * b

        base_row = wid * rows_pt
        for g in range(ngrp):
            pltpu.sync_copy(src_h.at[pl.ds(base_row + g * G, G)], midx)
            pltpu.sync_copy(dst_h.at[pl.ds(base_row + g * G, G)], didx)
            start_gathers(0, xxa, gsa)

            @pl.loop(0, G // 2)
            def _(k):
                b0 = 2 * k
                b1 = b0 + 1
                wait_gathers(b0, xxa, gsa)
                start_gathers(b1, xxb, gsb)

                @pl.when(k > 0)
                def _():
                    wait_scatter(ora, ssa)

                compute(xxa, ora)
                pltpu.async_copy(ora, acc.at[didx.at[b0]], ssa, add=True)

                wait_gathers(b1, xxb, gsb)

                @pl.when(k < G // 2 - 1)
                def _():
                    start_gathers(b0 + 2, xxa, gsa)

                @pl.when(k > 0)
                def _():
                    wait_scatter(orb, ssb)

                compute(xxb, orb)
                pltpu.async_copy(orb, acc.at[didx.at[b1]], ssb, add=True)

            wait_scatter(ora, ssa)
            wait_scatter(orb, ssb)

        plsc.subcore_barrier()
        pltpu.sync_copy(acc.at[pl.ds(sid * rpt, rpt)],
                        out_h.at[cid, pl.ds(sid * rpt, rpt)])

    return edge_kernel(table, src2, dst2, zeros)


def _prep_table(feats):
    n = feats.shape[0]
    r = 1000

    def body(f_ref, t_ref):
        f = f_ref[...]
        qn = jnp.sum(f * f, axis=1) - f[:, F - 1] * f[:, F - 1]
        qb = jnp.broadcast_to(qn[:, None], (r, TW - F))
        t_ref[...] = jnp.concatenate([f, qb], axis=1).astype(jnp.bfloat16)

    return pl.pallas_call(
        body,
        grid=(n // r,),
        in_specs=[pl.BlockSpec((r, F), lambda i: (i, 0))],
        out_specs=pl.BlockSpec((r, TW), lambda i: (i, 0)),
        out_shape=jax.ShapeDtypeStruct((n, TW), jnp.bfloat16),
    )(feats)


def _combine(parts, feats, wm, wn, last):
    n = feats.shape[0]
    r = 1000

    def body(p_ref, f_ref, wm_ref, wn_ref, *o_refs):
        s = p_ref[0] + p_ref[1]
        num = s[:, 0:F]
        deg = s[:, F]
        f = f_ref[...]
        agg = num / jnp.maximum(deg, DEG_EPS)[:, None]
        out = (jnp.dot(agg, wm_ref[...], preferred_element_type=jnp.float32)
               + jnp.dot(f, wn_ref[...], preferred_element_type=jnp.float32))
        if last:
            o_refs[0][...] = out
        else:
            f2 = jnp.maximum(out, 0.0)
            qn = jnp.sum(f2 * f2, axis=1) - f2[:, F - 1] * f2[:, F - 1]
            o_refs[0][...] = f2
            qb = jnp.broadcast_to(qn[:, None], (r, TW - F))
            o_refs[1][...] = jnp.concatenate(
                [f2, qb], axis=1).astype(jnp.bfloat16)

    if last:
        out_shape = [jax.ShapeDtypeStruct((n, F), jnp.float32)]
        out_specs = [pl.BlockSpec((r, F), lambda i: (i, 0))]
    else:
        out_shape = [jax.ShapeDtypeStruct((n, F), jnp.float32),
                     jax.ShapeDtypeStruct((n, TW), jnp.bfloat16)]
        out_specs = [pl.BlockSpec((r, F), lambda i: (i, 0)),
                     pl.BlockSpec((r, TW), lambda i: (i, 0))]

    res = pl.pallas_call(
        body,
        grid=(n // r,),
        in_specs=[
            pl.BlockSpec((NC, r, C), lambda i: (0, i, 0)),
            pl.BlockSpec((r, F), lambda i: (i, 0)),
            pl.BlockSpec((F, F), lambda i: (0, 0)),
            pl.BlockSpec((F, F), lambda i: (0, 0)),
        ],
        out_specs=out_specs,
        out_shape=out_shape,
    )(parts, feats, wm, wn)
    return res[0] if last else res


_UNPACK_PERM = np.concatenate(
    [np.concatenate([np.arange(16) * 2 + 32 * c,
                     np.arange(16) * 2 + 1 + 32 * c]) for c in range(4)])


def kernel(x, edge_index, W_msg, W_node):
    src2 = edge_index[0].reshape(-1, BE)
    dst2 = edge_index[1].reshape(-1, BE)
    midx2 = jnp.concatenate([src2, dst2], axis=1)
    n_layers = W_msg.shape[0]
    feats = x[:, :F]
    table = _prep_table(feats)
    wm_perm = W_msg[:, _UNPACK_PERM, :]
    zeros = jnp.zeros((x.shape[0] // NS, C), jnp.float32)
    for layer in range(n_layers):
        parts = _edge_pass(table, midx2, dst2, zeros)
        last = layer == n_layers - 1
        if last:
            return _combine(parts, feats,
                            wm_perm[layer], W_node[layer], True)
        feats, table = _combine(parts, feats,
                                wm_perm[layer], W_node[layer], False)


# speedup vs baseline: 1.0188x; 1.0188x over previous
import functools

import numpy as np

import jax
import jax.numpy as jnp
from jax import lax
from jax.experimental import pallas as pl
from jax.experimental.pallas import tpu as pltpu
from jax.experimental.pallas import tpu_sc as plsc

F = 128
C = 144
TW = 160
NC = 2
NS = 16
LANES = 16
BE = 40
G = 50
EPS = 1e-9
DEG_EPS = 1e-6
_IL = plsc.PackFormat.INTERLEAVED


def _edge_pass(table, src2, dst2, zeros):
    n = table.shape[0]
    nrows = src2.shape[0]
    n_tiles = NC * NS
    rows_pt = nrows // n_tiles
    ngrp = rows_pt // G
    rpt = n // NS

    mesh = plsc.VectorSubcoreMesh(
        core_axis_name="c", subcore_axis_name="s",
        num_cores=NC, num_subcores=NS)

    @functools.partial(
        pl.kernel,
        out_type=jax.ShapeDtypeStruct((NC, n, C), jnp.float32),
        mesh=mesh,
        scratch_types=[
            pltpu.VMEM((G, BE), jnp.int32),
            pltpu.VMEM((G, BE), jnp.int32),
            pltpu.VMEM((BE, TW), jnp.bfloat16),
            pltpu.VMEM((BE, TW), jnp.bfloat16),
            pltpu.VMEM((BE, TW), jnp.bfloat16),
            pltpu.VMEM((BE, TW), jnp.bfloat16),
            pltpu.VMEM((BE, C), jnp.float32),
            pltpu.VMEM((BE, C), jnp.float32),
            pltpu.VMEM_SHARED((n, C), jnp.float32),
            pltpu.SemaphoreType.DMA,
            pltpu.SemaphoreType.DMA,
            pltpu.SemaphoreType.DMA,
            pltpu.SemaphoreType.DMA,
        ],
        compiler_params=pltpu.CompilerParams(
            use_tc_tiling_on_sc=False, needs_layout_passes=False),
    )
    def edge_kernel(table_h, src_h, dst_h, zeros_h, out_h,
                    sidx, didx, xja, xjb, xia, xib, ora, orb, acc,
                    gsa, gsb, ssa, ssb):
        cid = lax.axis_index("c")
        sid = lax.axis_index("s")
        wid = sid * NC + cid

        lane = lax.iota(jnp.int32, LANES)
        unit = jnp.where(lane == 0, 1.0, 0.0).astype(jnp.float32)

        pltpu.sync_copy(zeros_h, acc.at[pl.ds(sid * rpt, rpt)])
        plsc.subcore_barrier()

        @pl.loop(0, BE)
        def _(ed):
            ora[ed, pl.ds(F, LANES)] = unit
            orb[ed, pl.ds(F, LANES)] = unit

        def start_gathers(b, xj, xi, sem):
            pltpu.async_copy(table_h.at[sidx.at[b]], xj, sem)
            pltpu.async_copy(table_h.at[didx.at[b]], xi, sem)

        def wait_gathers(b, xj, xi, sem):
            pltpu.make_async_copy(table_h.at[sidx.at[b]], xj, sem).wait()
            pltpu.make_async_copy(table_h.at[didx.at[b]], xi, sem).wait()

        def wait_scatter(orow, sem):
            pltpu.make_async_copy(orow, acc.at[didx.at[0]], sem).wait()

        def compute(xj, xi, orow):
            @plsc.parallel_loop(0, BE, unroll=4)
            def _(ed):
                xjc = [xj[ed, pl.ds(c * 32, 32)] for c in range(4)]
                xic = [xi[ed, pl.ds(c * 32, 32)] for c in range(4)]
                p = [xic[c] * xjc[c] for c in range(4)]
                s = (p[0] + p[1]) + (p[2] + p[3])
                lo, hi = plsc.unpack(s, format=_IL)
                dot = jnp.sum(lo + hi)
                dv = jnp.full((LANES,), dot, jnp.float32)
                nprod = xi[ed, pl.ds(F, 32)] * xj[ed, pl.ds(F, 32)]
                den, _ = plsc.unpack(nprod, format=_IL)
                q = dv * dv / jnp.maximum(den, EPS)
                wv = jnp.exp(q - 1.0)
                for c in range(4):
                    a, b = plsc.unpack(xjc[c], format=_IL)
                    orow[ed, pl.ds(c * 32, LANES)] = wv * a
                    orow[ed, pl.ds(c * 32 + LANES, LANES)] = wv * b

        base_row = wid * rows_pt
        for g in range(ngrp):
            pltpu.sync_copy(src_h.at[pl.ds(base_row + g * G, G)], sidx)
            pltpu.sync_copy(dst_h.at[pl.ds(base_row + g * G, G)], didx)
            start_gathers(0, xja, xia, gsa)

            @pl.loop(0, G // 2)
            def _(k):
                b0 = 2 * k
                b1 = b0 + 1
                wait_gathers(b0, xja, xia, gsa)
                start_gathers(b1, xjb, xib, gsb)

                @pl.when(k > 0)
                def _():
                    wait_scatter(ora, ssa)

                compute(xja, xia, ora)
                pltpu.async_copy(ora, acc.at[didx.at[b0]], ssa, add=True)

                wait_gathers(b1, xjb, xib, gsb)

                @pl.when(k < G // 2 - 1)
                def _():
                    start_gathers(b0 + 2, xja, xia, gsa)

                @pl.when(k > 0)
                def _():
                    wait_scatter(orb, ssb)

                compute(xjb, xib, orb)
                pltpu.async_copy(orb, acc.at[didx.at[b1]], ssb, add=True)

            wait_scatter(ora, ssa)
            wait_scatter(orb, ssb)

        plsc.subcore_barrier()
        pltpu.sync_copy(acc.at[pl.ds(sid * rpt, rpt)],
                        out_h.at[cid, pl.ds(sid * rpt, rpt)])

    return edge_kernel(table, src2, dst2, zeros)


def _prep_table(feats):
    n = feats.shape[0]
    r = 1000

    def body(f_ref, t_ref):
        f = f_ref[...]
        qn = jnp.sum(f * f, axis=1) - f[:, F - 1] * f[:, F - 1]
        qb = jnp.broadcast_to(qn[:, None], (r, TW - F))
        t_ref[...] = jnp.concatenate([f, qb], axis=1).astype(jnp.bfloat16)

    return pl.pallas_call(
        body,
        grid=(n // r,),
        in_specs=[pl.BlockSpec((r, F), lambda i: (i, 0))],
        out_specs=pl.BlockSpec((r, TW), lambda i: (i, 0)),
        out_shape=jax.ShapeDtypeStruct((n, TW), jnp.bfloat16),
    )(feats)


def _combine(parts, feats, wm, wn, last):
    n = feats.shape[0]
    r = 1000

    def body(p_ref, f_ref, wm_ref, wn_ref, *o_refs):
        s = p_ref[0] + p_ref[1]
        num = s[:, 0:F]
        deg = s[:, F]
        f = f_ref[...]
        agg = num / jnp.maximum(deg, DEG_EPS)[:, None]
        out = (jnp.dot(agg, wm_ref[...], preferred_element_type=jnp.float32)
               + jnp.dot(f, wn_ref[...], preferred_element_type=jnp.float32))
        if last:
            o_refs[0][...] = out
        else:
            f2 = jnp.maximum(out, 0.0)
            qn = jnp.sum(f2 * f2, axis=1) - f2[:, F - 1] * f2[:, F - 1]
            o_refs[0][...] = f2
            qb = jnp.broadcast_to(qn[:, None], (r, TW - F))
            o_refs[1][...] = jnp.concatenate(
                [f2, qb], axis=1).astype(jnp.bfloat16)

    if last:
        out_shape = [jax.ShapeDtypeStruct((n, F), jnp.float32)]
        out_specs = [pl.BlockSpec((r, F), lambda i: (i, 0))]
    else:
        out_shape = [jax.ShapeDtypeStruct((n, F), jnp.float32),
                     jax.ShapeDtypeStruct((n, TW), jnp.bfloat16)]
        out_specs = [pl.BlockSpec((r, F), lambda i: (i, 0)),
                     pl.BlockSpec((r, TW), lambda i: (i, 0))]

    res = pl.pallas_call(
        body,
        grid=(n // r,),
        in_specs=[
            pl.BlockSpec((NC, r, C), lambda i: (0, i, 0)),
            pl.BlockSpec((r, F), lambda i: (i, 0)),
            pl.BlockSpec((F, F), lambda i: (0, 0)),
            pl.BlockSpec((F, F), lambda i: (0, 0)),
        ],
        out_specs=out_specs,
        out_shape=out_shape,
    )(parts, feats, wm, wn)
    return res[0] if last else res


_UNPACK_PERM = np.concatenate(
    [np.concatenate([np.arange(16) * 2 + 32 * c,
                     np.arange(16) * 2 + 1 + 32 * c]) for c in range(4)])


def kernel(x, edge_index, W_msg, W_node):
    src2 = edge_index[0].reshape(-1, BE)
    dst2 = edge_index[1].reshape(-1, BE)
    n_layers = W_msg.shape[0]
    feats = x[:, :F]
    table = _prep_table(feats)
    wm_perm = W_msg[:, _UNPACK_PERM, :]
    zeros = jnp.zeros((x.shape[0] // NS, C), jnp.float32)
    for layer in range(n_layers):
        parts = _edge_pass(table, src2, dst2, zeros)
        last = layer == n_layers - 1
        if last:
            return _combine(parts, feats,
                            wm_perm[layer], W_node[layer], True)
        feats, table = _combine(parts, feats,
                                wm_perm[layer], W_node[layer], False)


# speedup vs baseline: 1.2484x; 1.2253x over previous
import functools

import numpy as np

import jax
import jax.numpy as jnp
from jax import lax
from jax.experimental import pallas as pl
from jax.experimental.pallas import tpu as pltpu
from jax.experimental.pallas import tpu_sc as plsc

F = 128
C = 144
TW = 160
NC = 2
NS = 16
LANES = 16
BE = 40
G = 50
EPS = 1e-9
DEG_EPS = 1e-6
_IL = plsc.PackFormat.INTERLEAVED


def _edge_pass(table, src2, dst2, zeros):
    n = table.shape[0]
    nrows = src2.shape[0]
    n_tiles = NC * NS
    rows_pt = nrows // n_tiles
    ngrp = rows_pt // G
    rpt = n // NS

    mesh = plsc.VectorSubcoreMesh(
        core_axis_name="c", subcore_axis_name="s",
        num_cores=NC, num_subcores=NS)

    @functools.partial(
        pl.kernel,
        out_type=jax.ShapeDtypeStruct((NC, n, C), jnp.float32),
        mesh=mesh,
        scratch_types=[
            pltpu.VMEM((G, BE), jnp.int32),
            pltpu.VMEM((G, BE), jnp.int32),
            pltpu.VMEM((BE, TW), jnp.bfloat16),
            pltpu.VMEM((BE, TW), jnp.bfloat16),
            pltpu.VMEM((BE, TW), jnp.bfloat16),
            pltpu.VMEM((BE, TW), jnp.bfloat16),
            pltpu.VMEM((BE, C), jnp.float32),
            pltpu.VMEM((BE, C), jnp.float32),
            pltpu.VMEM_SHARED((n, C), jnp.float32),
            pltpu.SemaphoreType.DMA,
            pltpu.SemaphoreType.DMA,
            pltpu.SemaphoreType.DMA,
            pltpu.SemaphoreType.DMA,
        ],
        compiler_params=pltpu.CompilerParams(
            use_tc_tiling_on_sc=False, needs_layout_passes=False),
    )
    def edge_kernel(table_h, src_h, dst_h, zeros_h, out_h,
                    sidx, didx, xja, xjb, xia, xib, ora, orb, acc,
                    gsa, gsb, ssa, ssb):
        cid = lax.axis_index("c")
        sid = lax.axis_index("s")
        wid = sid * NC + cid

        lane = lax.iota(jnp.int32, LANES)
        unit = jnp.where(lane == 0, 1.0, 0.0).astype(jnp.float32)

        pltpu.sync_copy(zeros_h, acc.at[pl.ds(sid * rpt, rpt)])
        plsc.subcore_barrier()

        @pl.loop(0, BE)
        def _(ed):
            ora[ed, pl.ds(F, LANES)] = unit
            orb[ed, pl.ds(F, LANES)] = unit

        def start_gathers(b, xj, xi, sem):
            pltpu.async_copy(table_h.at[sidx.at[b]], xj, sem)
            pltpu.async_copy(table_h.at[didx.at[b]], xi, sem)

        def wait_gathers(b, xj, xi, sem):
            pltpu.make_async_copy(table_h.at[sidx.at[b]], xj, sem).wait()
            pltpu.make_async_copy(table_h.at[didx.at[b]], xi, sem).wait()

        def wait_scatter(orow, sem):
            pltpu.make_async_copy(orow, acc.at[didx.at[0]], sem).wait()

        def compute(xj, xi, orow):
            @plsc.parallel_loop(0, BE, unroll=4)
            def _(ed):
                xjc = [xj[ed, pl.ds(c * 32, 32)] for c in range(4)]
                xic = [xi[ed, pl.ds(c * 32, 32)] for c in range(4)]
                p = [xic[c] * xjc[c] for c in range(4)]
                s = (p[0] + p[1]) + (p[2] + p[3])
                lo, hi = plsc.unpack(s, format=_IL)
                dot = jnp.sum(lo + hi)
                dv = jnp.full((LANES,), dot, jnp.float32)
                nprod = xi[ed, pl.ds(F, 32)] * xj[ed, pl.ds(F, 32)]
                den, _ = plsc.unpack(nprod, format=_IL)
                q = dv * dv / jnp.maximum(den, EPS)
                wv = jnp.exp(q - 1.0)
                for c in range(4):
                    a, b = plsc.unpack(xjc[c], format=_IL)
                    orow[ed, pl.ds(c * 32, LANES)] = wv * a
                    orow[ed, pl.ds(c * 32 + LANES, LANES)] = wv * b

        base_row = wid * rows_pt
        for g in range(ngrp):
            pltpu.sync_copy(src_h.at[pl.ds(base_row + g * G, G)], sidx)
            pltpu.sync_copy(dst_h.at[pl.ds(base_row + g * G, G)], didx)
            start_gathers(0, xja, xia, gsa)

            @pl.loop(0, G // 2)
            def _(k):
                b0 = 2 * k
                b1 = b0 + 1
                start_gathers(b1, xjb, xib, gsb)
                wait_gathers(b0, xja, xia, gsa)

                @pl.when(k > 0)
                def _():
                    wait_scatter(ora, ssa)

                compute(xja, xia, ora)
                pltpu.async_copy(ora, acc.at[didx.at[b0]], ssa, add=True)

                wait_gathers(b1, xjb, xib, gsb)

                @pl.when(k < G // 2 - 1)
                def _():
                    start_gathers(b0 + 2, xja, xia, gsa)

                @pl.when(k > 0)
                def _():
                    wait_scatter(orb, ssb)

                compute(xjb, xib, orb)
                pltpu.async_copy(orb, acc.at[didx.at[b1]], ssb, add=True)

            wait_scatter(ora, ssa)
            wait_scatter(orb, ssb)

        plsc.subcore_barrier()
        pltpu.sync_copy(acc.at[pl.ds(sid * rpt, rpt)],
                        out_h.at[cid, pl.ds(sid * rpt, rpt)])

    return edge_kernel(table, src2, dst2, zeros)


def _prep_table(feats):
    n = feats.shape[0]
    r = 1000

    def body(f_ref, t_ref):
        f = f_ref[...]
        qn = jnp.sum(f * f, axis=1) - f[:, F - 1] * f[:, F - 1]
        qb = jnp.broadcast_to(qn[:, None], (r, TW - F))
        t_ref[...] = jnp.concatenate([f, qb], axis=1).astype(jnp.bfloat16)

    return pl.pallas_call(
        body,
        grid=(n // r,),
        in_specs=[pl.BlockSpec((r, F), lambda i: (i, 0))],
        out_specs=pl.BlockSpec((r, TW), lambda i: (i, 0)),
        out_shape=jax.ShapeDtypeStruct((n, TW), jnp.bfloat16),
    )(feats)


def _combine(parts, feats, wm, wn, last):
    n = feats.shape[0]
    r = 1000

    def body(p_ref, f_ref, wm_ref, wn_ref, *o_refs):
        s = p_ref[0] + p_ref[1]
        num = s[:, 0:F]
        deg = s[:, F]
        f = f_ref[...]
        agg = num / jnp.maximum(deg, DEG_EPS)[:, None]
        out = (jnp.dot(agg, wm_ref[...], preferred_element_type=jnp.float32)
               + jnp.dot(f, wn_ref[...], preferred_element_type=jnp.float32))
        if last:
            o_refs[0][...] = out
        else:
            f2 = jnp.maximum(out, 0.0)
            qn = jnp.sum(f2 * f2, axis=1) - f2[:, F - 1] * f2[:, F - 1]
            o_refs[0][...] = f2
            qb = jnp.broadcast_to(qn[:, None], (r, TW - F))
            o_refs[1][...] = jnp.concatenate(
                [f2, qb], axis=1).astype(jnp.bfloat16)

    if last:
        out_shape = [jax.ShapeDtypeStruct((n, F), jnp.float32)]
        out_specs = [pl.BlockSpec((r, F), lambda i: (i, 0))]
    else:
        out_shape = [jax.ShapeDtypeStruct((n, F), jnp.float32),
                     jax.ShapeDtypeStruct((n, TW), jnp.bfloat16)]
        out_specs = [pl.BlockSpec((r, F), lambda i: (i, 0)),
                     pl.BlockSpec((r, TW), lambda i: (i, 0))]

    res = pl.pallas_call(
        body,
        grid=(n // r,),
        in_specs=[
            pl.BlockSpec((NC, r, C), lambda i: (0, i, 0)),
            pl.BlockSpec((r, F), lambda i: (i, 0)),
            pl.BlockSpec((F, F), lambda i: (0, 0)),
            pl.BlockSpec((F, F), lambda i: (0, 0)),
        ],
        out_specs=out_specs,
        out_shape=out_shape,
    )(parts, feats, wm, wn)
    return res[0] if last else res


_UNPACK_PERM = np.concatenate(
    [np.concatenate([np.arange(16) * 2 + 32 * c,
                     np.arange(16) * 2 + 1 + 32 * c]) for c in range(4)])


def kernel(x, edge_index, W_msg, W_node):
    src2 = edge_index[0].reshape(-1, BE)
    dst2 = edge_index[1].reshape(-1, BE)
    n_layers = W_msg.shape[0]
    feats = x[:, :F]
    table = _prep_table(feats)
    wm_perm = W_msg[:, _UNPACK_PERM, :]
    zeros = jnp.zeros((x.shape[0] // NS, C), jnp.float32)
    for layer in range(n_layers):
        parts = _edge_pass(table, src2, dst2, zeros)
        last = layer == n_layers - 1
        if last:
            return _combine(parts, feats,
                            wm_perm[layer], W_node[layer], True)
        feats, table = _combine(parts, feats,
                                wm_perm[layer], W_node[layer], False)


# speedup vs baseline: 1.2615x; 1.0105x over previous
import functools

import numpy as np

import jax
import jax.numpy as jnp
from jax import lax
from jax.experimental import pallas as pl
from jax.experimental.pallas import tpu as pltpu
from jax.experimental.pallas import tpu_sc as plsc

F = 128
C = 144
TW = 160
NC = 2
NS = 16
LANES = 16
BE = 40
G = 50
EPS = 1e-9
DEG_EPS = 1e-6
_IL = plsc.PackFormat.INTERLEAVED


def _edge_pass(table, src2, dst2, zeros):
    n = table.shape[0]
    nrows = src2.shape[0]
    n_tiles = NC * NS
    rows_pt = nrows // n_tiles
    ngrp = rows_pt // G
    rpt = n // NS

    mesh = plsc.VectorSubcoreMesh(
        core_axis_name="c", subcore_axis_name="s",
        num_cores=NC, num_subcores=NS)

    @functools.partial(
        pl.kernel,
        out_type=jax.ShapeDtypeStruct((NC, n, C), jnp.float32),
        mesh=mesh,
        scratch_types=[
            pltpu.VMEM((G, BE), jnp.int32),
            pltpu.VMEM((G, BE), jnp.int32),
            pltpu.VMEM((BE, TW), jnp.bfloat16),
            pltpu.VMEM((BE, TW), jnp.bfloat16),
            pltpu.VMEM((BE, TW), jnp.bfloat16),
            pltpu.VMEM((BE, TW), jnp.bfloat16),
            pltpu.VMEM((BE, C), jnp.float32),
            pltpu.VMEM((BE, C), jnp.float32),
            pltpu.VMEM_SHARED((n, C), jnp.float32),
            pltpu.SemaphoreType.DMA,
            pltpu.SemaphoreType.DMA,
            pltpu.SemaphoreType.DMA,
            pltpu.SemaphoreType.DMA,
        ],
        compiler_params=pltpu.CompilerParams(
            use_tc_tiling_on_sc=False, needs_layout_passes=False),
    )
    def edge_kernel(table_h, src_h, dst_h, zeros_h, out_h,
                    sidx, didx, xja, xjb, xia, xib, ora, orb, acc,
                    gsa, gsb, ssa, ssb):
        cid = lax.axis_index("c")
        sid = lax.axis_index("s")
        wid = sid * NC + cid

        lane = lax.iota(jnp.int32, LANES)
        unit = jnp.where(lane == 0, 1.0, 0.0).astype(jnp.float32)

        pltpu.sync_copy(zeros_h, acc.at[pl.ds(sid * rpt, rpt)])
        plsc.subcore_barrier()

        @pl.loop(0, BE)
        def _(ed):
            ora[ed, pl.ds(F, LANES)] = unit
            orb[ed, pl.ds(F, LANES)] = unit

        def start_gathers(b, xj, xi, sem):
            pltpu.async_copy(table_h.at[sidx.at[b]], xj, sem)
            pltpu.async_copy(table_h.at[didx.at[b]], xi, sem)

        def wait_gathers(b, xj, xi, sem):
            pltpu.make_async_copy(table_h.at[sidx.at[b]], xj, sem).wait()
            pltpu.make_async_copy(table_h.at[didx.at[b]], xi, sem).wait()

        def wait_scatter(orow, sem):
            pltpu.make_async_copy(orow, acc.at[didx.at[0]], sem).wait()

        def compute(xj, xi, orow):
            @plsc.parallel_loop(0, BE, unroll=4)
            def _(ed):
                xjc = [xj[ed, pl.ds(c * 32, 32)] for c in range(4)]
                xic = [xi[ed, pl.ds(c * 32, 32)] for c in range(4)]
                p = [xic[c] * xjc[c] for c in range(4)]
                s = (p[0] + p[1]) + (p[2] + p[3])
                lo, hi = plsc.unpack(s, format=_IL)
                dot = jnp.sum(lo + hi)
                dv = jnp.full((LANES,), dot, jnp.float32)
                nprod = xi[ed, pl.ds(F, 32)] * xj[ed, pl.ds(F, 32)]
                den, _ = plsc.unpack(nprod, format=_IL)
                q = dv * dv / jnp.maximum(den, EPS)
                wv = jnp.exp(q - 1.0)
                for c in range(4):
                    a, b = plsc.unpack(xjc[c], format=_IL)
                    orow[ed, pl.ds(c * 32, LANES)] = wv * a
                    orow[ed, pl.ds(c * 32 + LANES, LANES)] = wv * b

        base_row = wid * rows_pt
        for g in range(ngrp):
            pltpu.sync_copy(src_h.at[pl.ds(base_row + g * G, G)], sidx)
            pltpu.sync_copy(dst_h.at[pl.ds(base_row + g * G, G)], didx)
            start_gathers(0, xja, xia, gsa)

            @pl.loop(0, G // 2)
            def _(k):
                b0 = 2 * k
                b1 = b0 + 1
                start_gathers(b1, xjb, xib, gsb)
                wait_gathers(b0, xja, xia, gsa)

                @pl.when(k > 0)
                def _():
                    wait_scatter(ora, ssa)

                compute(xja, xia, ora)

                @pl.when(k < G // 2 - 1)
                def _():
                    start_gathers(b0 + 2, xja, xia, gsa)

                pltpu.async_copy(ora, acc.at[didx.at[b0]], ssa, add=True)
                wait_gathers(b1, xjb, xib, gsb)

                @pl.when(k > 0)
                def _():
                    wait_scatter(orb, ssb)

                compute(xjb, xib, orb)
                pltpu.async_copy(orb, acc.at[didx.at[b1]], ssb, add=True)

            wait_scatter(ora, ssa)
            wait_scatter(orb, ssb)

        plsc.subcore_barrier()
        pltpu.sync_copy(acc.at[pl.ds(sid * rpt, rpt)],
                        out_h.at[cid, pl.ds(sid * rpt, rpt)])

    return edge_kernel(table, src2, dst2, zeros)


def _prep_table(feats):
    n = feats.shape[0]
    r = 1000

    def body(f_ref, t_ref):
        f = f_ref[...]
        qn = jnp.sum(f * f, axis=1) - f[:, F - 1] * f[:, F - 1]
        qb = jnp.broadcast_to(qn[:, None], (r, TW - F))
        t_ref[...] = jnp.concatenate([f, qb], axis=1).astype(jnp.bfloat16)

    return pl.pallas_call(
        body,
        grid=(n // r,),
        in_specs=[pl.BlockSpec((r, F), lambda i: (i, 0))],
        out_specs=pl.BlockSpec((r, TW), lambda i: (i, 0)),
        out_shape=jax.ShapeDtypeStruct((n, TW), jnp.bfloat16),
    )(feats)


def _combine(parts, feats, wm, wn, last):
    n = feats.shape[0]
    r = 1000

    def body(p_ref, f_ref, wm_ref, wn_ref, *o_refs):
        s = p_ref[0] + p_ref[1]
        num = s[:, 0:F]
        deg = s[:, F]
        f = f_ref[...]
        agg = num / jnp.maximum(deg, DEG_EPS)[:, None]
        out = (jnp.dot(agg, wm_ref[...], preferred_element_type=jnp.float32)
               + jnp.dot(f, wn_ref[...], preferred_element_type=jnp.float32))
        if last:
            o_refs[0][...] = out
        else:
            f2 = jnp.maximum(out, 0.0)
            qn = jnp.sum(f2 * f2, axis=1) - f2[:, F - 1] * f2[:, F - 1]
            o_refs[0][...] = f2
            qb = jnp.broadcast_to(qn[:, None], (r, TW - F))
            o_refs[1][...] = jnp.concatenate(
                [f2, qb], axis=1).astype(jnp.bfloat16)

    if last:
        out_shape = [jax.ShapeDtypeStruct((n, F), jnp.float32)]
        out_specs = [pl.BlockSpec((r, F), lambda i: (i, 0))]
    else:
        out_shape = [jax.ShapeDtypeStruct((n, F), jnp.float32),
                     jax.ShapeDtypeStruct((n, TW), jnp.bfloat16)]
        out_specs = [pl.BlockSpec((r, F), lambda i: (i, 0)),
                     pl.BlockSpec((r, TW), lambda i: (i, 0))]

    res = pl.pallas_call(
        body,
        grid=(n // r,),
        in_specs=[
            pl.BlockSpec((NC, r, C), lambda i: (0, i, 0)),
            pl.BlockSpec((r, F), lambda i: (i, 0)),
            pl.BlockSpec((F, F), lambda i: (0, 0)),
            pl.BlockSpec((F, F), lambda i: (0, 0)),
        ],
        out_specs=out_specs,
        out_shape=out_shape,
    )(parts, feats, wm, wn)
    return res[0] if last else res


_UNPACK_PERM = np.concatenate(
    [np.concatenate([np.arange(16) * 2 + 32 * c,
                     np.arange(16) * 2 + 1 + 32 * c]) for c in range(4)])


def kernel(x, edge_index, W_msg, W_node):
    src2 = edge_index[0].reshape(-1, BE)
    dst2 = edge_index[1].reshape(-1, BE)
    n_layers = W_msg.shape[0]
    feats = x[:, :F]
    table = _prep_table(feats)
    wm_perm = W_msg[:, _UNPACK_PERM, :]
    zeros = jnp.zeros((x.shape[0] // NS, C), jnp.float32)
    for layer in range(n_layers):
        parts = _edge_pass(table, src2, dst2, zeros)
        last = layer == n_layers - 1
        if last:
            return _combine(parts, feats,
                            wm_perm[layer], W_node[layer], True)
        feats, table = _combine(parts, feats,
                                wm_perm[layer], W_node[layer], False)


# speedup vs baseline: 1.2831x; 1.0171x over previous
import functools

import numpy as np

import jax
import jax.numpy as jnp
from jax import lax
from jax.experimental import pallas as pl
from jax.experimental.pallas import tpu as pltpu
from jax.experimental.pallas import tpu_sc as plsc

F = 128
C = 144
TW = 160
NC = 2
NS = 16
LANES = 16
BE = 40
G = 50
EPS = 1e-9
DEG_EPS = 1e-6
_IL = plsc.PackFormat.INTERLEAVED


def _edge_pass(table, src2, dst2, zeros):
    n = table.shape[0]
    nrows = src2.shape[0]
    n_tiles = NC * NS
    rows_pt = nrows // n_tiles
    ngrp = rows_pt // G
    rpt = n // NS

    mesh = plsc.VectorSubcoreMesh(
        core_axis_name="c", subcore_axis_name="s",
        num_cores=NC, num_subcores=NS)

    @functools.partial(
        pl.kernel,
        out_type=jax.ShapeDtypeStruct((NC, n, C), jnp.float32),
        mesh=mesh,
        scratch_types=[
            pltpu.VMEM((2, G, BE), jnp.int32),
            pltpu.VMEM((2, G, BE), jnp.int32),
            pltpu.VMEM((BE, TW), jnp.bfloat16),
            pltpu.VMEM((BE, TW), jnp.bfloat16),
            pltpu.VMEM((BE, TW), jnp.bfloat16),
            pltpu.VMEM((BE, TW), jnp.bfloat16),
            pltpu.VMEM((BE, C), jnp.float32),
            pltpu.VMEM((BE, C), jnp.float32),
            pltpu.VMEM_SHARED((n, C), jnp.float32),
            pltpu.SemaphoreType.DMA,
            pltpu.SemaphoreType.DMA,
            pltpu.SemaphoreType.DMA,
            pltpu.SemaphoreType.DMA,
            pltpu.SemaphoreType.DMA,
        ],
        compiler_params=pltpu.CompilerParams(
            use_tc_tiling_on_sc=False, needs_layout_passes=False),
    )
    def edge_kernel(table_h, src_h, dst_h, zeros_h, out_h,
                    sidx2, didx2, xja, xjb, xia, xib, ora, orb, acc,
                    gsa, gsb, ssa, ssb, gsi):
        cid = lax.axis_index("c")
        sid = lax.axis_index("s")
        wid = sid * NC + cid

        lane = lax.iota(jnp.int32, LANES)
        unit = jnp.where(lane == 0, 1.0, 0.0).astype(jnp.float32)

        pltpu.sync_copy(zeros_h, acc.at[pl.ds(sid * rpt, rpt)])
        plsc.subcore_barrier()

        @pl.loop(0, BE)
        def _(ed):
            ora[ed, pl.ds(F, LANES)] = unit
            orb[ed, pl.ds(F, LANES)] = unit

        def wait_scatter(orow, didx, sem):
            pltpu.make_async_copy(orow, acc.at[didx.at[0]], sem).wait()

        def compute(xj, xi, orow):
            @plsc.parallel_loop(0, BE, unroll=4)
            def _(ed):
                xjc = [xj[ed, pl.ds(c * 32, 32)] for c in range(4)]
                xic = [xi[ed, pl.ds(c * 32, 32)] for c in range(4)]
                p = [xic[c] * xjc[c] for c in range(4)]
                s = (p[0] + p[1]) + (p[2] + p[3])
                lo, hi = plsc.unpack(s, format=_IL)
                dot = jnp.sum(lo + hi)
                dv = jnp.full((LANES,), dot, jnp.float32)
                nprod = xi[ed, pl.ds(F, 32)] * xj[ed, pl.ds(F, 32)]
                den, _ = plsc.unpack(nprod, format=_IL)
                q = dv * dv / jnp.maximum(den, EPS)
                wv = jnp.exp(q - 1.0)
                for c in range(4):
                    a, b = plsc.unpack(xjc[c], format=_IL)
                    orow[ed, pl.ds(c * 32, LANES)] = wv * a
                    orow[ed, pl.ds(c * 32 + LANES, LANES)] = wv * b

        base_row = wid * rows_pt
        pltpu.sync_copy(src_h.at[pl.ds(base_row, G)], sidx2.at[0])
        pltpu.sync_copy(dst_h.at[pl.ds(base_row, G)], didx2.at[0])
        for g in range(ngrp):
            sidx = sidx2.at[g % 2]
            didx = didx2.at[g % 2]

            def start_gathers(b, xj, xi, sem):
                pltpu.async_copy(table_h.at[sidx.at[b]], xj, sem)
                pltpu.async_copy(table_h.at[didx.at[b]], xi, sem)

            def wait_gathers(b, xj, xi, sem):
                pltpu.make_async_copy(table_h.at[sidx.at[b]], xj, sem).wait()
                pltpu.make_async_copy(table_h.at[didx.at[b]], xi, sem).wait()

            start_gathers(0, xja, xia, gsa)
            if g + 1 < ngrp:
                nxt = base_row + (g + 1) * G
                pltpu.async_copy(src_h.at[pl.ds(nxt, G)],
                                 sidx2.at[(g + 1) % 2], gsi)
                pltpu.async_copy(dst_h.at[pl.ds(nxt, G)],
                                 didx2.at[(g + 1) % 2], gsi)

            @pl.loop(0, G // 2)
            def _(k):
                b0 = 2 * k
                b1 = b0 + 1
                start_gathers(b1, xjb, xib, gsb)
                wait_gathers(b0, xja, xia, gsa)

                @pl.when(k > 0)
                def _():
                    wait_scatter(ora, didx, ssa)

                compute(xja, xia, ora)

                @pl.when(k < G // 2 - 1)
                def _():
                    start_gathers(b0 + 2, xja, xia, gsa)

                pltpu.async_copy(ora, acc.at[didx.at[b0]], ssa, add=True)
                wait_gathers(b1, xjb, xib, gsb)

                @pl.when(k > 0)
                def _():
                    wait_scatter(orb, didx, ssb)

                compute(xjb, xib, orb)
                pltpu.async_copy(orb, acc.at[didx.at[b1]], ssb, add=True)

            wait_scatter(ora, didx, ssa)
            wait_scatter(orb, didx, ssb)
            if g + 1 < ngrp:
                nxt = base_row + (g + 1) * G
                pltpu.make_async_copy(src_h.at[pl.ds(nxt, G)],
                                      sidx2.at[(g + 1) % 2], gsi).wait()
                pltpu.make_async_copy(dst_h.at[pl.ds(nxt, G)],
                                      didx2.at[(g + 1) % 2], gsi).wait()

        plsc.subcore_barrier()
        pltpu.sync_copy(acc.at[pl.ds(sid * rpt, rpt)],
                        out_h.at[cid, pl.ds(sid * rpt, rpt)])

    return edge_kernel(table, src2, dst2, zeros)


def _prep_table(feats):
    n = feats.shape[0]
    r = 1000

    def body(f_ref, t_ref):
        f = f_ref[...]
        qn = jnp.sum(f * f, axis=1) - f[:, F - 1] * f[:, F - 1]
        qb = jnp.broadcast_to(qn[:, None], (r, TW - F))
        t_ref[...] = jnp.concatenate([f, qb], axis=1).astype(jnp.bfloat16)

    return pl.pallas_call(
        body,
        grid=(n // r,),
        in_specs=[pl.BlockSpec((r, F), lambda i: (i, 0))],
        out_specs=pl.BlockSpec((r, TW), lambda i: (i, 0)),
        out_shape=jax.ShapeDtypeStruct((n, TW), jnp.bfloat16),
    )(feats)


def _combine(parts, feats, wm, wn, last):
    n = feats.shape[0]
    r = 1000

    def body(p_ref, f_ref, wm_ref, wn_ref, *o_refs):
        s = p_ref[0] + p_ref[1]
        num = s[:, 0:F]
        deg = s[:, F]
        f = f_ref[...]
        agg = num / jnp.maximum(deg, DEG_EPS)[:, None]
        out = (jnp.dot(agg, wm_ref[...], preferred_element_type=jnp.float32)
               + jnp.dot(f, wn_ref[...], preferred_element_type=jnp.float32))
        if last:
            o_refs[0][...] = out
        else:
            f2 = jnp.maximum(out, 0.0)
            qn = jnp.sum(f2 * f2, axis=1) - f2[:, F - 1] * f2[:, F - 1]
            o_refs[0][...] = f2
            qb = jnp.broadcast_to(qn[:, None], (r, TW - F))
            o_refs[1][...] = jnp.concatenate(
                [f2, qb], axis=1).astype(jnp.bfloat16)

    if last:
        out_shape = [jax.ShapeDtypeStruct((n, F), jnp.float32)]
        out_specs = [pl.BlockSpec((r, F), lambda i: (i, 0))]
    else:
        out_shape = [jax.ShapeDtypeStruct((n, F), jnp.float32),
                     jax.ShapeDtypeStruct((n, TW), jnp.bfloat16)]
        out_specs = [pl.BlockSpec((r, F), lambda i: (i, 0)),
                     pl.BlockSpec((r, TW), lambda i: (i, 0))]

    res = pl.pallas_call(
        body,
        grid=(n // r,),
        in_specs=[
            pl.BlockSpec((NC, r, C), lambda i: (0, i, 0)),
            pl.BlockSpec((r, F), lambda i: (i, 0)),
            pl.BlockSpec((F, F), lambda i: (0, 0)),
            pl.BlockSpec((F, F), lambda i: (0, 0)),
        ],
        out_specs=out_specs,
        out_shape=out_shape,
    )(parts, feats, wm, wn)
    return res[0] if last else res


_UNPACK_PERM = np.concatenate(
    [np.concatenate([np.arange(16) * 2 + 32 * c,
                     np.arange(16) * 2 + 1 + 32 * c]) for c in range(4)])


def kernel(x, edge_index, W_msg, W_node):
    src2 = edge_index[0].reshape(-1, BE)
    dst2 = edge_index[1].reshape(-1, BE)
    n_layers = W_msg.shape[0]
    feats = x[:, :F]
    table = _prep_table(feats)
    wm_perm = W_msg[:, _UNPACK_PERM, :]
    zeros = jnp.zeros((x.shape[0] // NS, C), jnp.float32)
    for layer in range(n_layers):
        parts = _edge_pass(table, src2, dst2, zeros)
        last = layer == n_layers - 1
        if last:
            return _combine(parts, feats,
                            wm_perm[layer], W_node[layer], True)
        feats, table = _combine(parts, feats,
                                wm_perm[layer], W_node[layer], False)
